# Initial kernel scaffold; baseline (speedup 1.0000x reference)
import jax
import jax.numpy as jnp
from jax.experimental import pallas as pl


def kernel(x, edge_index, W_enc, b_enc, W_pred, b_pred):
    raise NotImplementedError("write your pallas kernel here")


# speedup vs baseline: 12.1036x; 12.1036x over previous
import jax
import jax.numpy as jnp
from jax import lax
from jax.experimental import pallas as pl
from jax.experimental.pallas import tpu as pltpu
from jax.experimental.pallas import tpu_sc as plsc

N = 10000
F = 128
E = 320000
NT = 4

NP = 10240
NC, NS = 2, 16
NW = NC * NS
K = 64
CH = 158
EW = CH * K
EP = NW * EW
RPT = NP // NS
L = 16
DR = NP // F


def _sc_agg_body(table_hbm, mask_hbm, idx_hbm, agg_out, deg_out,
                 idx_v, rows_v, mask_v, deg_v, iota_v, agg_sh, deg_sh,
                 sem, sem_i):
    c = lax.axis_index("c")
    s = lax.axis_index("s")
    wid = c * NS + s

    pltpu.sync_copy(mask_hbm, mask_v)
    for k in range(DR // L):
        iota_v[pl.ds(k * L, L)] = lax.iota(jnp.int32, L) + (k * L)

    zv = jnp.zeros((L,), jnp.float32)

    def zrow(i, carry):
        for j in range(F // L):
            rows_v[0, i, pl.ds(j * L, L)] = zv
        return carry

    lax.fori_loop(0, K, zrow, 0)

    def zdeg(i, carry):
        for j in range(F // L):
            deg_v[i, pl.ds(j * L, L)] = zv
        return carry

    lax.fori_loop(0, DR, zdeg, 0)

    base = s * RPT
    for k in range(RPT // K):
        pltpu.sync_copy(rows_v.at[0], agg_sh.at[pl.ds(base + k * K, K)])

    @pl.when(s < DR // 8)
    def _():
        pltpu.sync_copy(rows_v.at[0].at[pl.ds(0, 8)],
                        deg_sh.at[pl.ds(s * 8, 8)])

    plsc.subcore_barrier()

    pltpu.sync_copy(idx_hbm.at[wid].at[0], idx_v.at[0])
    pltpu.async_copy(table_hbm.at[idx_v.at[0].at[0]], rows_v.at[0],
                     sem.at[0])

    def chunk(j, carry):
        b = lax.rem(j, 2)
        nb = 1 - b

        @pl.when(j < CH - 1)
        def _():
            pltpu.async_copy(idx_hbm.at[wid].at[j + 1], idx_v.at[nb],
                             sem_i.at[nb])

        for v in range(K // L):
            sv = idx_v[b, 0, pl.ds(v * L, L)]
            dv = idx_v[b, 1, pl.ds(v * L, L)]
            mv = plsc.load_gather(mask_v, [sv])
            plsc.addupdate_scatter(deg_v, [dv >> 7, dv & 127], mv)
        pltpu.make_async_copy(table_hbm.at[idx_v.at[b].at[0]], rows_v.at[b],
                              sem.at[b]).wait()
        pltpu.sync_copy(rows_v.at[b], agg_sh.at[idx_v.at[b].at[1]], add=True)

        @pl.when(j < CH - 1)
        def _():
            pltpu.make_async_copy(idx_hbm.at[wid].at[j + 1], idx_v.at[nb],
                                  sem_i.at[nb]).wait()
            pltpu.async_copy(table_hbm.at[idx_v.at[nb].at[0]],
                             rows_v.at[nb], sem.at[nb])

        return carry

    lax.fori_loop(0, CH, chunk, 0)

    pltpu.sync_copy(deg_v, deg_sh.at[iota_v], add=True)
    plsc.subcore_barrier()

    for k in range(RPT // K):
        pltpu.sync_copy(agg_sh.at[pl.ds(base + k * K, K)],
                        agg_out.at[c].at[pl.ds(base + k * K, K)])

    @pl.when(s < DR // 8)
    def _():
        pltpu.sync_copy(deg_sh.at[pl.ds(s * 8, 8)],
                        deg_out.at[c].at[pl.ds(s * 8, 8)])


_sc_agg = pl.kernel(
    _sc_agg_body,
    out_type=(
        jax.ShapeDtypeStruct((NC, NP, F), jnp.float32),
        jax.ShapeDtypeStruct((NC, DR, F), jnp.float32),
    ),
    mesh=plsc.VectorSubcoreMesh(core_axis_name="c", subcore_axis_name="s"),
    compiler_params=pltpu.CompilerParams(needs_layout_passes=False),
    scratch_types=[
        pltpu.VMEM((2, 2, K), jnp.int32),
        pltpu.VMEM((2, K, F), jnp.float32),
        pltpu.VMEM((NP,), jnp.float32),
        pltpu.VMEM((DR, F), jnp.float32),
        pltpu.VMEM((DR,), jnp.int32),
        pltpu.VMEM_SHARED((NP, F), jnp.float32),
        pltpu.VMEM_SHARED((DR, F), jnp.float32),
        pltpu.SemaphoreType.DMA((2,)),
        pltpu.SemaphoreType.DMA((2,)),
    ],
)


def _prep_body(x_ref, m_ref, t_ref):
    t_ref[...] = x_ref[...] * m_ref[...]


def _prep(x_pad, m_pad):
    blk = NP // 4
    return pl.pallas_call(
        _prep_body,
        grid=(4,),
        in_specs=[
            pl.BlockSpec((blk, F), lambda g: (g, 0)),
            pl.BlockSpec((blk, 1), lambda g: (g, 0)),
        ],
        out_specs=pl.BlockSpec((blk, F), lambda g: (g, 0)),
        out_shape=jax.ShapeDtypeStruct((NP, F), jnp.float32),
    )(x_pad, m_pad)


def _finish_body(p_ref, d_ref, m_ref, we_ref, be_ref, wp_ref, bp_ref, o_ref):
    p = p_ref[...]
    a = p[0] + p[1]
    d = d_ref[...]
    dcol = d[0] + d[1]
    scale = m_ref[...] / jnp.maximum(dcol, 1.0)
    h = a * scale
    h = jnp.maximum(jnp.dot(h, we_ref[...],
                            preferred_element_type=jnp.float32,
                            precision=lax.Precision.HIGHEST) + be_ref[...], 0.0)
    wp = wp_ref[...]
    bp = bp_ref[...]
    for t in range(NT):
        h = jnp.maximum(jnp.dot(h, wp,
                                preferred_element_type=jnp.float32,
                                precision=lax.Precision.HIGHEST) + bp, 0.0)
        o_ref[t] = h


def _finish(parts, degs, m_pad, W_enc, b_enc, W_pred, b_pred):
    blk = 1000
    grid = N // blk
    return pl.pallas_call(
        _finish_body,
        grid=(grid,),
        in_specs=[
            pl.BlockSpec((NC, blk, F), lambda g: (0, g, 0)),
            pl.BlockSpec((NC, blk, 1), lambda g: (0, g, 0)),
            pl.BlockSpec((blk, 1), lambda g: (g, 0)),
            pl.BlockSpec((F, F), lambda g: (0, 0)),
            pl.BlockSpec((1, F), lambda g: (0, 0)),
            pl.BlockSpec((F, F), lambda g: (0, 0)),
            pl.BlockSpec((1, F), lambda g: (0, 0)),
        ],
        out_specs=pl.BlockSpec((NT, blk, F), lambda g: (0, g, 0)),
        out_shape=jax.ShapeDtypeStruct((NT, N, F), jnp.float32),
    )(parts, degs, m_pad, W_enc, b_enc, W_pred, b_pred)


def kernel(x, edge_index, W_enc, b_enc, W_pred, b_pred):
    mask = jax.random.bernoulli(jax.random.key(42), 0.5, (N,))
    m_col = mask.astype(jnp.float32)[:, None]
    m_pad = jnp.pad(m_col, ((0, NP - N), (0, 0)))
    x_pad = jnp.pad(x, ((0, NP - N), (0, 0)))
    table = _prep(x_pad, m_pad)

    ei = jnp.pad(edge_index, ((0, 0), (0, EP - E)), constant_values=N)
    idx = jnp.transpose(ei.reshape(2, NW, CH, K), (1, 2, 0, 3))

    parts, degs = _sc_agg(table, m_pad[:, 0], idx)

    return _finish(parts, degs.reshape(NC, NP, 1), m_pad[:N], W_enc,
                   b_enc.reshape(1, F), W_pred, b_pred.reshape(1, F))
